# initial kernel scaffold (unmeasured)
import jax
import jax.numpy as jnp
from jax import lax
from jax.experimental import pallas as pl
from jax.experimental.pallas import tpu as pltpu

N_DEV = 4
B = 2
SQ = 512
DM = 768
SKV_SH = 512
H_SH = 8
DH = 64
HD_SH = H_SH * DH
BLK = 64
NRES = 4

bf16 = jnp.bfloat16
f32 = jnp.float32


def kernel(x, Wq, K_ext, V_ext, Wo):
    def body(x_ref, wq_ref, k_ref, v_ref, wo_ref, out_ref,
             ksend, vsend, kgath, vgath, ogath,
             ksend_sems, vsend_sems, krecv_sems, vrecv_sems,
             osend_sems, orecv_sems):
        my = lax.axis_index("i")

        for d in range(N_DEV):
            ksend[d] = k_ref[:, :, d * H_SH:(d + 1) * H_SH, :].astype(bf16)
            vsend[d] = v_ref[:, :, d * H_SH:(d + 1) * H_SH, :].astype(bf16)

        for m in range(N_DEV):
            for d in range(N_DEV):
                if d == m:
                    continue

                @pl.when(my == m)
                def _(m=m, d=d):
                    for src, gath, ssem, rsem in (
                        (ksend, kgath, ksend_sems, krecv_sems),
                        (vsend, vgath, vsend_sems, vrecv_sems),
                    ):
                        rdma = pltpu.make_async_remote_copy(
                            src_ref=src.at[d],
                            dst_ref=gath.at[m],
                            send_sem=ssem.at[d],
                            recv_sem=rsem.at[m],
                            device_id=(d,),
                            device_id_type=pl.DeviceIdType.MESH,
                        )
                        rdma.start()

        for m in range(N_DEV):
            @pl.when(my == m)
            def _(m=m):
                kgath[m] = ksend[m]
                vgath[m] = vsend[m]

        wq = wq_ref[...].astype(bf16)
        qs = []
        for b in range(B):
            qf = jnp.dot(x_ref[b].astype(bf16), wq,
                         preferred_element_type=f32)
            qs.append((qf * 0.125).astype(bf16).reshape(SQ, H_SH, DH))

        for s in range(N_DEV):
            @pl.when(s != my)
            def _(s=s):
                for src, gath, ssem, rsem in (
                    (ksend, kgath, ksend_sems, krecv_sems),
                    (vsend, vgath, vsend_sems, vrecv_sems),
                ):
                    rdma = pltpu.make_async_remote_copy(
                        src_ref=src.at[s],
                        dst_ref=gath.at[s],
                        send_sem=ssem.at[s],
                        recv_sem=rsem.at[s],
                        device_id=(s,),
                        device_id_type=pl.DeviceIdType.MESH,
                    )
                    rdma.wait_recv()

        for d in range(N_DEV):
            @pl.when(d != my)
            def _(d=d):
                for src, gath, ssem, rsem in (
                    (ksend, kgath, ksend_sems, krecv_sems),
                    (vsend, vgath, vsend_sems, vrecv_sems),
                ):
                    rdma = pltpu.make_async_remote_copy(
                        src_ref=src.at[d],
                        dst_ref=gath.at[d],
                        send_sem=ssem.at[d],
                        recv_sem=rsem.at[d],
                        device_id=(d,),
                        device_id_type=pl.DeviceIdType.MESH,
                    )
                    rdma.wait_send()

        wo = wo_ref[...].astype(bf16)
        partials = []
        for b in range(B):
            q_b = qs[b]
            k_b = kgath[:, b].reshape(N_DEV * SKV_SH, H_SH, DH)
            v_b = vgath[:, b].reshape(N_DEV * SKV_SH, H_SH, DH)
            lo, hi = [], []
            for r in range(NRES):
                qr = jnp.concatenate(
                    [q_b[BLK * r:BLK * (r + 1)],
                     q_b[BLK * (r + 4):BLK * (r + 5)]], axis=0)
                kr = jnp.concatenate(
                    [k_b[BLK * (r + 4 * t):BLK * (r + 4 * t) + BLK]
                     for t in range(8)], axis=0)
                vr = jnp.concatenate(
                    [v_b[BLK * (r + 4 * t):BLK * (r + 4 * t) + BLK]
                     for t in range(8)], axis=0)
                scores = lax.dot_general(
                    qr, kr, (((2,), (2,)), ((1,), (1,))),
                    preferred_element_type=f32)
                mx = jnp.max(scores, axis=-1, keepdims=True)
                w = jnp.exp(scores - mx)
                w = w / jnp.sum(w, axis=-1, keepdims=True)
                ctx = lax.dot_general(
                    w.astype(bf16), vr, (((2,), (0,)), ((0,), (1,))),
                    preferred_element_type=f32)
                ctx = jnp.swapaxes(ctx, 0, 1).reshape(2 * BLK, HD_SH)
                ctx = ctx.astype(bf16)
                lo.append(ctx[:BLK])
                hi.append(ctx[BLK:])
            ctx_full = jnp.concatenate(lo + hi, axis=0)
            partials.append(
                jnp.dot(ctx_full, wo, preferred_element_type=f32).astype(bf16))

        for m in range(N_DEV):
            @pl.when(my == m)
            def _(m=m):
                for b in range(B):
                    ogath[m, b] = partials[b]

        for m in range(N_DEV):
            for d in range(N_DEV):
                if d == m:
                    continue

                @pl.when(my == m)
                def _(m=m, d=d):
                    rdma = pltpu.make_async_remote_copy(
                        src_ref=ogath.at[m],
                        dst_ref=ogath.at[m],
                        send_sem=osend_sems.at[d],
                        recv_sem=orecv_sems.at[m],
                        device_id=(d,),
                        device_id_type=pl.DeviceIdType.MESH,
                    )
                    rdma.start()

        for s in range(N_DEV):
            @pl.when(s != my)
            def _(s=s):
                rdma = pltpu.make_async_remote_copy(
                    src_ref=ogath.at[s],
                    dst_ref=ogath.at[s],
                    send_sem=osend_sems.at[s],
                    recv_sem=orecv_sems.at[s],
                    device_id=(s,),
                    device_id_type=pl.DeviceIdType.MESH,
                )
                rdma.wait_recv()

        out_ref[...] = (ogath[0].astype(f32) + ogath[1].astype(f32)
                        + ogath[2].astype(f32) + ogath[3].astype(f32))

        for d in range(N_DEV):
            @pl.when(d != my)
            def _(d=d):
                rdma = pltpu.make_async_remote_copy(
                    src_ref=ogath.at[d],
                    dst_ref=ogath.at[d],
                    send_sem=osend_sems.at[d],
                    recv_sem=orecv_sems.at[d],
                    device_id=(d,),
                    device_id_type=pl.DeviceIdType.MESH,
                )
                rdma.wait_send()

    out_shape = jax.ShapeDtypeStruct((B, SQ, DM), f32)
    return pl.pallas_call(
        body,
        out_shape=out_shape,
        in_specs=[pl.BlockSpec(memory_space=pltpu.VMEM)] * 5,
        out_specs=pl.BlockSpec(memory_space=pltpu.VMEM),
        scratch_shapes=[
            pltpu.VMEM((N_DEV, B, SKV_SH, H_SH, DH), bf16),
            pltpu.VMEM((N_DEV, B, SKV_SH, H_SH, DH), bf16),
            pltpu.VMEM((N_DEV, B, SKV_SH, H_SH, DH), bf16),
            pltpu.VMEM((N_DEV, B, SKV_SH, H_SH, DH), bf16),
            pltpu.VMEM((N_DEV, B, SQ, DM), bf16),
            pltpu.SemaphoreType.DMA((N_DEV,)),
            pltpu.SemaphoreType.DMA((N_DEV,)),
            pltpu.SemaphoreType.DMA((N_DEV,)),
            pltpu.SemaphoreType.DMA((N_DEV,)),
            pltpu.SemaphoreType.DMA((N_DEV,)),
            pltpu.SemaphoreType.DMA((N_DEV,)),
        ],
        compiler_params=pltpu.CompilerParams(
            vmem_limit_bytes=64 * 1024 * 1024,
        ),
    )(x, Wq, K_ext, V_ext, Wo)


# baseline (device time: 172060 ns/iter reference)
import jax
import jax.numpy as jnp
from jax import lax
from jax.experimental import pallas as pl
from jax.experimental.pallas import tpu as pltpu

N_DEV = 4
B = 2
SQ = 512
DM = 768
SKV_SH = 512
H_SH = 8
DH = 64
HD_SH = H_SH * DH
BLK = 64
NRES = 4

bf16 = jnp.bfloat16
f32 = jnp.float32


def _pack(K_ext, V_ext):
    def body(k_ref, v_ref, kp_ref, vp_ref):
        for d in range(N_DEV):
            ks = k_ref[:, :, d * H_SH:(d + 1) * H_SH, :]
            vs = v_ref[:, :, d * H_SH:(d + 1) * H_SH, :]
            kp_ref[d] = ks.reshape(B, SKV_SH, HD_SH).astype(bf16)
            vp_ref[d] = vs.reshape(B, SKV_SH, HD_SH).astype(bf16)

    shp = jax.ShapeDtypeStruct((N_DEV, B, SKV_SH, HD_SH), bf16)
    return pl.pallas_call(
        body,
        out_shape=(shp, shp),
        in_specs=[pl.BlockSpec(memory_space=pltpu.VMEM)] * 2,
        out_specs=(pl.BlockSpec(memory_space=pltpu.VMEM),) * 2,
        compiler_params=pltpu.CompilerParams(
            vmem_limit_bytes=64 * 1024 * 1024,
        ),
    )(K_ext, V_ext)


def kernel(x, Wq, K_ext, V_ext, Wo):
    kpk, vpk = _pack(K_ext, V_ext)

    def body(x_ref, wq_ref, kpk_ref, vpk_ref, wo_ref, out_ref,
             kgath, vgath, ogath, cbuf,
             ksend_sems, vsend_sems, krecv_sems, vrecv_sems,
             osend_sems, orecv_sems):
        my = lax.axis_index("i")

        for m in range(N_DEV):
            for d in range(N_DEV):
                if d == m:
                    continue

                @pl.when(my == m)
                def _(m=m, d=d):
                    for src, gath, ssem, rsem in (
                        (kpk_ref, kgath, ksend_sems, krecv_sems),
                        (vpk_ref, vgath, vsend_sems, vrecv_sems),
                    ):
                        rdma = pltpu.make_async_remote_copy(
                            src_ref=src.at[d],
                            dst_ref=gath.at[m],
                            send_sem=ssem.at[d],
                            recv_sem=rsem.at[m],
                            device_id=(d,),
                            device_id_type=pl.DeviceIdType.MESH,
                        )
                        rdma.start()

        for m in range(N_DEV):
            @pl.when(my == m)
            def _(m=m):
                kgath[m] = kpk_ref[m]
                vgath[m] = vpk_ref[m]

        wq = wq_ref[...].astype(bf16)
        qs = []
        for b in range(B):
            qf = jnp.dot(x_ref[b].astype(bf16), wq,
                         preferred_element_type=f32)
            qs.append((qf * 0.125).astype(bf16))

        for s in range(N_DEV):
            @pl.when(s != my)
            def _(s=s):
                for src, gath, ssem, rsem in (
                    (kpk_ref, kgath, ksend_sems, krecv_sems),
                    (vpk_ref, vgath, vsend_sems, vrecv_sems),
                ):
                    rdma = pltpu.make_async_remote_copy(
                        src_ref=src.at[s],
                        dst_ref=gath.at[s],
                        send_sem=ssem.at[s],
                        recv_sem=rsem.at[s],
                        device_id=(s,),
                        device_id_type=pl.DeviceIdType.MESH,
                    )
                    rdma.wait_recv()

        for d in range(N_DEV):
            @pl.when(d != my)
            def _(d=d):
                for src, gath, ssem, rsem in (
                    (kpk_ref, kgath, ksend_sems, krecv_sems),
                    (vpk_ref, vgath, vsend_sems, vrecv_sems),
                ):
                    rdma = pltpu.make_async_remote_copy(
                        src_ref=src.at[d],
                        dst_ref=gath.at[d],
                        send_sem=ssem.at[d],
                        recv_sem=rsem.at[d],
                        device_id=(d,),
                        device_id_type=pl.DeviceIdType.MESH,
                    )
                    rdma.wait_send()

        for b in range(B):
            q_b = qs[b]
            k_b = kgath[:, b].reshape(N_DEV * SKV_SH, HD_SH)
            v_b = vgath[:, b].reshape(N_DEV * SKV_SH, HD_SH)
            for r in range(NRES):
                qr = jnp.concatenate(
                    [q_b[BLK * r:BLK * (r + 1)],
                     q_b[BLK * (r + 4):BLK * (r + 5)]], axis=0)
                kr = jnp.concatenate(
                    [k_b[BLK * (r + 4 * t):BLK * (r + 4 * t) + BLK]
                     for t in range(8)], axis=0)
                vr = jnp.concatenate(
                    [v_b[BLK * (r + 4 * t):BLK * (r + 4 * t) + BLK]
                     for t in range(8)], axis=0)
                qr3 = qr.reshape(2 * BLK, H_SH, DH)
                kr3 = kr.reshape(8 * BLK, H_SH, DH)
                vr3 = vr.reshape(8 * BLK, H_SH, DH)
                scores = lax.dot_general(
                    qr3, kr3, (((2,), (2,)), ((1,), (1,))),
                    preferred_element_type=f32)
                mx = jnp.max(scores, axis=-1, keepdims=True)
                w = jnp.exp(scores - mx)
                w = w / jnp.sum(w, axis=-1, keepdims=True)
                ctx = lax.dot_general(
                    w.astype(bf16), vr3, (((2,), (0,)), ((0,), (1,))),
                    preferred_element_type=f32)
                ctx = jnp.swapaxes(ctx, 0, 1).reshape(2 * BLK, HD_SH)
                ctx = ctx.astype(bf16)
                cbuf[b, BLK * r:BLK * (r + 1)] = ctx[:BLK]
                cbuf[b, BLK * (r + 4):BLK * (r + 5)] = ctx[BLK:]

        wo = wo_ref[...].astype(bf16)
        partials = [
            jnp.dot(cbuf[b], wo, preferred_element_type=f32).astype(bf16)
            for b in range(B)
        ]
        for m in range(N_DEV):
            @pl.when(my == m)
            def _(m=m):
                for b in range(B):
                    ogath[m, b] = partials[b]

        for m in range(N_DEV):
            for d in range(N_DEV):
                if d == m:
                    continue

                @pl.when(my == m)
                def _(m=m, d=d):
                    rdma = pltpu.make_async_remote_copy(
                        src_ref=ogath.at[m],
                        dst_ref=ogath.at[m],
                        send_sem=osend_sems.at[d],
                        recv_sem=orecv_sems.at[m],
                        device_id=(d,),
                        device_id_type=pl.DeviceIdType.MESH,
                    )
                    rdma.start()

        for s in range(N_DEV):
            @pl.when(s != my)
            def _(s=s):
                rdma = pltpu.make_async_remote_copy(
                    src_ref=ogath.at[s],
                    dst_ref=ogath.at[s],
                    send_sem=osend_sems.at[s],
                    recv_sem=orecv_sems.at[s],
                    device_id=(s,),
                    device_id_type=pl.DeviceIdType.MESH,
                )
                rdma.wait_recv()

        out_ref[...] = (ogath[0].astype(f32) + ogath[1].astype(f32)
                        + ogath[2].astype(f32) + ogath[3].astype(f32))

        for d in range(N_DEV):
            @pl.when(d != my)
            def _(d=d):
                rdma = pltpu.make_async_remote_copy(
                    src_ref=ogath.at[d],
                    dst_ref=ogath.at[d],
                    send_sem=osend_sems.at[d],
                    recv_sem=orecv_sems.at[d],
                    device_id=(d,),
                    device_id_type=pl.DeviceIdType.MESH,
                )
                rdma.wait_send()

    out_shape = jax.ShapeDtypeStruct((B, SQ, DM), f32)
    return pl.pallas_call(
        body,
        out_shape=out_shape,
        in_specs=[pl.BlockSpec(memory_space=pltpu.VMEM)] * 5,
        out_specs=pl.BlockSpec(memory_space=pltpu.VMEM),
        scratch_shapes=[
            pltpu.VMEM((N_DEV, B, SKV_SH, HD_SH), bf16),
            pltpu.VMEM((N_DEV, B, SKV_SH, HD_SH), bf16),
            pltpu.VMEM((N_DEV, B, SQ, DM), bf16),
            pltpu.VMEM((B, SQ, HD_SH), bf16),
            pltpu.SemaphoreType.DMA((N_DEV,)),
            pltpu.SemaphoreType.DMA((N_DEV,)),
            pltpu.SemaphoreType.DMA((N_DEV,)),
            pltpu.SemaphoreType.DMA((N_DEV,)),
            pltpu.SemaphoreType.DMA((N_DEV,)),
            pltpu.SemaphoreType.DMA((N_DEV,)),
        ],
        compiler_params=pltpu.CompilerParams(
            vmem_limit_bytes=64 * 1024 * 1024,
        ),
    )(x, Wq, kpk, vpk, Wo)


# device time: 126242 ns/iter; 1.3629x vs baseline; 1.3629x over previous
import jax
import jax.numpy as jnp
from jax import lax
from jax.experimental import pallas as pl
from jax.experimental.pallas import tpu as pltpu

N_DEV = 4
B = 2
SQ = 512
DM = 768
SKV_SH = 512
H_SH = 8
DH = 64
HD_SH = H_SH * DH
BLK = 64
NRES = 4

bf16 = jnp.bfloat16
f32 = jnp.float32


def _pack(K_ext, V_ext):
    def body(k_ref, v_ref, kp_ref, vp_ref):
        for d in range(N_DEV):
            kp_ref[d] = k_ref[:, :, HD_SH * d:HD_SH * (d + 1)].astype(bf16)
            vp_ref[d] = v_ref[:, :, HD_SH * d:HD_SH * (d + 1)].astype(bf16)

    shp = jax.ShapeDtypeStruct((N_DEV, B, SKV_SH, HD_SH), bf16)
    return pl.pallas_call(
        body,
        out_shape=(shp, shp),
        in_specs=[pl.BlockSpec(memory_space=pltpu.VMEM)] * 2,
        out_specs=(pl.BlockSpec(memory_space=pltpu.VMEM),) * 2,
        compiler_params=pltpu.CompilerParams(
            vmem_limit_bytes=64 * 1024 * 1024,
        ),
    )(K_ext.reshape(B, SKV_SH, N_DEV * HD_SH),
      V_ext.reshape(B, SKV_SH, N_DEV * HD_SH))


def kernel(x, Wq, K_ext, V_ext, Wo):
    kpk, vpk = _pack(K_ext, V_ext)

    def body(x_ref, wq_ref, kpk_ref, vpk_ref, wo_ref, out_ref,
             kgath, vgath, cbuf, pbuf, rsbuf, qred, agbuf,
             ksend_sems, vsend_sems, krecv_sems, vrecv_sems,
             rssend_sems, rsrecv_sems, agsend_sems, agrecv_sems):
        my = lax.axis_index("i")

        for m in range(N_DEV):
            for d in range(N_DEV):
                if d == m:
                    continue

                @pl.when(my == m)
                def _(m=m, d=d):
                    for src, gath, ssem, rsem in (
                        (kpk_ref, kgath, ksend_sems, krecv_sems),
                        (vpk_ref, vgath, vsend_sems, vrecv_sems),
                    ):
                        rdma = pltpu.make_async_remote_copy(
                            src_ref=src.at[d],
                            dst_ref=gath.at[m],
                            send_sem=ssem.at[d],
                            recv_sem=rsem.at[m],
                            device_id=(d,),
                            device_id_type=pl.DeviceIdType.MESH,
                        )
                        rdma.start()

        for m in range(N_DEV):
            @pl.when(my == m)
            def _(m=m):
                kgath[m] = kpk_ref[m]
                vgath[m] = vpk_ref[m]

        wq = wq_ref[...].astype(bf16)
        qs = []
        for b in range(B):
            qf = jnp.dot(x_ref[b].astype(bf16), wq,
                         preferred_element_type=f32)
            qs.append((qf * 0.125).astype(bf16))

        for s in range(N_DEV):
            @pl.when(s != my)
            def _(s=s):
                for src, gath, ssem, rsem in (
                    (kpk_ref, kgath, ksend_sems, krecv_sems),
                    (vpk_ref, vgath, vsend_sems, vrecv_sems),
                ):
                    rdma = pltpu.make_async_remote_copy(
                        src_ref=src.at[s],
                        dst_ref=gath.at[s],
                        send_sem=ssem.at[s],
                        recv_sem=rsem.at[s],
                        device_id=(s,),
                        device_id_type=pl.DeviceIdType.MESH,
                    )
                    rdma.wait_recv()

        for d in range(N_DEV):
            @pl.when(d != my)
            def _(d=d):
                for src, gath, ssem, rsem in (
                    (kpk_ref, kgath, ksend_sems, krecv_sems),
                    (vpk_ref, vgath, vsend_sems, vrecv_sems),
                ):
                    rdma = pltpu.make_async_remote_copy(
                        src_ref=src.at[d],
                        dst_ref=gath.at[d],
                        send_sem=ssem.at[d],
                        recv_sem=rsem.at[d],
                        device_id=(d,),
                        device_id_type=pl.DeviceIdType.MESH,
                    )
                    rdma.wait_send()

        for b in range(B):
            q_b = qs[b]
            k_b = kgath[:, b].reshape(N_DEV * SKV_SH, HD_SH)
            v_b = vgath[:, b].reshape(N_DEV * SKV_SH, HD_SH)
            for r in range(NRES):
                qr = jnp.concatenate(
                    [q_b[BLK * r:BLK * (r + 1)],
                     q_b[BLK * (r + 4):BLK * (r + 5)]], axis=0)
                kr = jnp.concatenate(
                    [k_b[BLK * (r + 4 * t):BLK * (r + 4 * t) + BLK]
                     for t in range(8)], axis=0)
                vr = jnp.concatenate(
                    [v_b[BLK * (r + 4 * t):BLK * (r + 4 * t) + BLK]
                     for t in range(8)], axis=0)
                qr3 = qr.reshape(2 * BLK, H_SH, DH)
                kr3 = kr.reshape(8 * BLK, H_SH, DH)
                vr3 = vr.reshape(8 * BLK, H_SH, DH)
                scores = lax.dot_general(
                    qr3, kr3, (((2,), (2,)), ((1,), (1,))),
                    preferred_element_type=f32)
                mx = jnp.max(scores, axis=-1, keepdims=True)
                w = jnp.exp(scores - mx)
                w = w / jnp.sum(w, axis=-1, keepdims=True)
                ctx = lax.dot_general(
                    w.astype(bf16), vr3, (((2,), (0,)), ((0,), (1,))),
                    preferred_element_type=f32)
                ctx = jnp.swapaxes(ctx, 0, 1).reshape(2 * BLK, HD_SH)
                ctx = ctx.astype(bf16)
                cbuf[b, BLK * r:BLK * (r + 1)] = ctx[:BLK]
                cbuf[b, BLK * (r + 4):BLK * (r + 5)] = ctx[BLK:]

        wo = wo_ref[...].astype(bf16)
        for b in range(B):
            pbuf[b] = jnp.dot(cbuf[b], wo,
                              preferred_element_type=f32).astype(bf16)

        QR = SQ // N_DEV
        for m in range(N_DEV):
            for d in range(N_DEV):
                if d == m:
                    continue

                @pl.when(my == m)
                def _(m=m, d=d):
                    rdma = pltpu.make_async_remote_copy(
                        src_ref=pbuf.at[:, QR * d:QR * (d + 1), :],
                        dst_ref=rsbuf.at[m],
                        send_sem=rssend_sems.at[d],
                        recv_sem=rsrecv_sems.at[m],
                        device_id=(d,),
                        device_id_type=pl.DeviceIdType.MESH,
                    )
                    rdma.start()

        for s in range(N_DEV):
            @pl.when(s != my)
            def _(s=s):
                rdma = pltpu.make_async_remote_copy(
                    src_ref=pbuf.at[:, QR * s:QR * (s + 1), :],
                    dst_ref=rsbuf.at[s],
                    send_sem=rssend_sems.at[s],
                    recv_sem=rsrecv_sems.at[s],
                    device_id=(s,),
                    device_id_type=pl.DeviceIdType.MESH,
                )
                rdma.wait_recv()

        for m in range(N_DEV):
            @pl.when(my == m)
            def _(m=m):
                acc = pbuf[:, QR * m:QR * (m + 1), :].astype(f32)
                for s in range(N_DEV):
                    if s != m:
                        acc = acc + rsbuf[s].astype(f32)
                out_ref[:, QR * m:QR * (m + 1), :] = acc
                qred[...] = acc.astype(bf16)

        for m in range(N_DEV):
            for d in range(N_DEV):
                if d == m:
                    continue

                @pl.when(my == m)
                def _(m=m, d=d):
                    rdma = pltpu.make_async_remote_copy(
                        src_ref=qred,
                        dst_ref=agbuf.at[m],
                        send_sem=agsend_sems.at[d],
                        recv_sem=agrecv_sems.at[m],
                        device_id=(d,),
                        device_id_type=pl.DeviceIdType.MESH,
                    )
                    rdma.start()

        for s in range(N_DEV):
            @pl.when(s != my)
            def _(s=s):
                rdma = pltpu.make_async_remote_copy(
                    src_ref=qred,
                    dst_ref=agbuf.at[s],
                    send_sem=agsend_sems.at[s],
                    recv_sem=agrecv_sems.at[s],
                    device_id=(s,),
                    device_id_type=pl.DeviceIdType.MESH,
                )
                rdma.wait_recv()
                out_ref[:, QR * s:QR * (s + 1), :] = agbuf[s].astype(f32)

        for d in range(N_DEV):
            @pl.when(d != my)
            def _(d=d):
                rs = pltpu.make_async_remote_copy(
                    src_ref=pbuf.at[:, QR * d:QR * (d + 1), :],
                    dst_ref=rsbuf.at[d],
                    send_sem=rssend_sems.at[d],
                    recv_sem=rsrecv_sems.at[d],
                    device_id=(d,),
                    device_id_type=pl.DeviceIdType.MESH,
                )
                rs.wait_send()
                ag = pltpu.make_async_remote_copy(
                    src_ref=qred,
                    dst_ref=agbuf.at[d],
                    send_sem=agsend_sems.at[d],
                    recv_sem=agrecv_sems.at[d],
                    device_id=(d,),
                    device_id_type=pl.DeviceIdType.MESH,
                )
                ag.wait_send()

    out_shape = jax.ShapeDtypeStruct((B, SQ, DM), f32)
    return pl.pallas_call(
        body,
        out_shape=out_shape,
        in_specs=[pl.BlockSpec(memory_space=pltpu.VMEM)] * 5,
        out_specs=pl.BlockSpec(memory_space=pltpu.VMEM),
        scratch_shapes=[
            pltpu.VMEM((N_DEV, B, SKV_SH, HD_SH), bf16),
            pltpu.VMEM((N_DEV, B, SKV_SH, HD_SH), bf16),
            pltpu.VMEM((B, SQ, HD_SH), bf16),
            pltpu.VMEM((B, SQ, DM), bf16),
            pltpu.VMEM((N_DEV, B, SQ // N_DEV, DM), bf16),
            pltpu.VMEM((B, SQ // N_DEV, DM), bf16),
            pltpu.VMEM((N_DEV, B, SQ // N_DEV, DM), bf16),
            pltpu.SemaphoreType.DMA((N_DEV,)),
            pltpu.SemaphoreType.DMA((N_DEV,)),
            pltpu.SemaphoreType.DMA((N_DEV,)),
            pltpu.SemaphoreType.DMA((N_DEV,)),
            pltpu.SemaphoreType.DMA((N_DEV,)),
            pltpu.SemaphoreType.DMA((N_DEV,)),
            pltpu.SemaphoreType.DMA((N_DEV,)),
            pltpu.SemaphoreType.DMA((N_DEV,)),
        ],
        compiler_params=pltpu.CompilerParams(
            vmem_limit_bytes=64 * 1024 * 1024,
        ),
    )(x, Wq, kpk, vpk, Wo)
